# Initial kernel scaffold; baseline (speedup 1.0000x reference)
import jax
import jax.numpy as jnp
from jax.experimental import pallas as pl


def kernel(x, Wg1, bg1, Wg2, bg2, W1, b1, W2, b2):
    raise NotImplementedError("write your pallas kernel here")


# speedup vs baseline: 2.4461x; 2.4461x over previous
import functools

import jax
import jax.numpy as jnp
from jax.experimental import pallas as pl
from jax.experimental.pallas import tpu as pltpu

N = 2048
D = 768
F = 1536
E = 8
TB = 256


def _moe_body(x_ref, wg1_ref, bg1_ref, wg2_ref, bg2_ref,
              w1_ref, b1_ref, w2_ref, b2_ref, out_ref, comb_ref):
    e = pl.program_id(0)
    t = pl.program_id(1)
    xb = x_ref[...]

    @pl.when(e == 0)
    def _gate():
        g1 = jnp.maximum(
            jnp.dot(xb, wg1_ref[...], preferred_element_type=jnp.float32)
            + bg1_ref[...], 0.0)
        logits = (jnp.dot(g1, wg2_ref[...], preferred_element_type=jnp.float32)
                  + bg2_ref[...])
        m = jnp.max(logits, axis=-1, keepdims=True)
        ex = jnp.exp(logits - m)
        p = ex / jnp.sum(ex, axis=-1, keepdims=True)
        iota = jax.lax.broadcasted_iota(jnp.int32, (TB, E), 1)
        w1 = jnp.max(p, axis=-1, keepdims=True)
        i1 = jnp.min(jnp.where(p >= w1, iota, E + 1), axis=-1, keepdims=True)
        p2 = jnp.where(iota == i1, -jnp.inf, p)
        w2 = jnp.max(p2, axis=-1, keepdims=True)
        i2 = jnp.min(jnp.where(p2 >= w2, iota, E + 1), axis=-1, keepdims=True)
        a = jnp.exp(w2 - w1)
        c1 = 1.0 / (1.0 + a)
        c2 = a * c1
        comb_ref[pl.ds(t * TB, TB), :] = (jnp.where(iota == i1, c1, 0.0)
                                          + jnp.where(iota == i2, c2, 0.0))
        out_ref[pl.ds(t * TB, TB), :] = jnp.zeros((TB, D), jnp.float32)

    h = (jnp.dot(xb, w1_ref[0], preferred_element_type=jnp.float32)
         + b1_ref[0])
    h = 0.5 * h * (1.0 + jax.lax.erf(h * (2.0 ** -0.5)))
    o = jnp.dot(h, w2_ref[0], preferred_element_type=jnp.float32) + b2_ref[0]
    cw = comb_ref[pl.ds(t * TB, TB), pl.ds(0, E)]
    iota_e = jax.lax.broadcasted_iota(jnp.int32, (TB, E), 1)
    ce = jnp.sum(jnp.where(iota_e == e, cw, 0.0), axis=-1, keepdims=True)
    out_ref[pl.ds(t * TB, TB), :] += ce * o


def _moe_dense(x_flat, Wg1, bg1, Wg2, bg2, W1, b1, W2, b2):
    T = N // TB
    return pl.pallas_call(
        _moe_body,
        grid=(E, T),
        in_specs=[
            pl.BlockSpec((TB, D), lambda e, t: (t, 0)),
            pl.BlockSpec((D, D // 2), lambda e, t: (0, 0)),
            pl.BlockSpec((1, D // 2), lambda e, t: (0, 0)),
            pl.BlockSpec((D // 2, E), lambda e, t: (0, 0)),
            pl.BlockSpec((1, E), lambda e, t: (0, 0)),
            pl.BlockSpec((1, D, F), lambda e, t: (e, 0, 0)),
            pl.BlockSpec((1, 1, F), lambda e, t: (e, 0, 0)),
            pl.BlockSpec((1, F, D), lambda e, t: (e, 0, 0)),
            pl.BlockSpec((1, 1, D), lambda e, t: (e, 0, 0)),
        ],
        out_specs=pl.BlockSpec((N, D), lambda e, t: (0, 0)),
        out_shape=jax.ShapeDtypeStruct((N, D), jnp.float32),
        scratch_shapes=[pltpu.VMEM((N, E), jnp.float32)],
    )(x_flat, Wg1, bg1.reshape(1, -1), Wg2, bg2.reshape(1, -1),
      W1, b1.reshape(E, 1, F), W2, b2.reshape(E, 1, D))


def kernel(x, Wg1, bg1, Wg2, bg2, W1, b1, W2, b2):
    B, S, _ = x.shape
    x_flat = x.reshape(-1, D)
    out = _moe_dense(x_flat, Wg1, bg1, Wg2, bg2, W1, b1, W2, b2)
    return out.reshape(B, S, D)
